# baseline (speedup 1.0000x reference)
import functools

import jax
import jax.numpy as jnp
from jax import lax
from jax.experimental import pallas as pl
from jax.experimental.pallas import tpu as pltpu
from jax.experimental.pallas import tpu_sc as plsc

N = 10000
D = 128
E = 320000
NC = 2
NS = 16
NW = NC * NS
EPW = E // NW
CHUNK = 104
NCHUNK = EPW // CHUNK
TAIL = EPW - NCHUNK * CHUNK
RPT = 624
ZR = 48

BLK = 1000
NB = N // BLK


def _seg_sum_body(x_hbm, src_hbm, dstm_hbm, dstt_hbm, out_hbm,
                  idx_s, idx_d, idx_dt, rows, acc,
                  sem_ip, sem_g0, sem_g1, sem_s0, sem_s1):
    c = lax.axis_index("c")
    s = lax.axis_index("s")
    wid = s * NC + c

    sem_g = (sem_g0, sem_g1)
    sem_s = (sem_s0, sem_s1)

    def gather_start(j, b):
        pltpu.async_copy(x_hbm.at[idx_s.at[pl.ds(j * CHUNK, CHUNK)]],
                         rows.at[b], sem_g[b])

    def gather_wait(j, b):
        pltpu.make_async_copy(x_hbm.at[idx_s.at[pl.ds(j * CHUNK, CHUNK)]],
                              rows.at[b], sem_g[b]).wait()

    def scatter_start(j, b):
        pltpu.async_copy(rows.at[b], acc.at[idx_d.at[j]], sem_s[b], add=True)

    def scatter_wait(j, b):
        pltpu.make_async_copy(rows.at[b], acc.at[idx_d.at[j]],
                              sem_s[b]).wait()

    ip_s = pltpu.async_copy(
        src_hbm.at[pl.ds(pl.multiple_of(wid * EPW, 8), EPW)], idx_s, sem_ip)
    ip_d = pltpu.async_copy(dstm_hbm.at[wid], idx_d, sem_ip)
    ip_t = pltpu.async_copy(dstt_hbm.at[wid], idx_dt, sem_ip)

    def zb(i, _):
        r = i // 8
        col = (i % 8) * 16
        rows[0, r, pl.ds(col, 16)] = jnp.zeros((16,), jnp.float32)
        return 0
    lax.fori_loop(0, ZR * 8, zb, 0)

    row0 = s * RPT

    def zoff(i):
        return pl.ds(pl.multiple_of(row0 + i * ZR, 8), ZR)

    def zc(i, _):
        pltpu.async_copy(rows.at[0, pl.ds(0, ZR)], acc.at[zoff(i)], sem_ip)
        return 0
    lax.fori_loop(0, RPT // ZR, zc, 0)

    @pl.when(s == NS - 1)
    def _():
        pltpu.async_copy(rows.at[0, pl.ds(0, 16)],
                         acc.at[pl.ds(N - 16, 16)], sem_ip)

    def zd(i, _):
        pltpu.make_async_copy(rows.at[0, pl.ds(0, ZR)], acc.at[zoff(i)],
                              sem_ip).wait()
        return 0
    lax.fori_loop(0, RPT // ZR, zd, 0)

    @pl.when(s == NS - 1)
    def _():
        pltpu.make_async_copy(rows.at[0, pl.ds(0, 16)],
                              acc.at[pl.ds(N - 16, 16)], sem_ip).wait()
    ip_s.wait()
    ip_d.wait()
    ip_t.wait()
    plsc.subcore_barrier()
    gather_start(0, 0)

    def body(i, _):
        j0 = i * 2
        j1 = j0 + 1
        gather_wait(j0, 0)

        @pl.when(i > 0)
        def _():
            scatter_wait(j0 - 1, 1)
        gather_start(j1, 1)
        scatter_start(j0, 0)
        gather_wait(j1, 1)
        scatter_wait(j0, 0)

        @pl.when(j0 + 2 < NCHUNK)
        def _():
            gather_start(j0 + 2, 0)
        scatter_start(j1, 1)
        return 0
    lax.fori_loop(0, NCHUNK // 2, body, 0)
    t0 = pl.multiple_of(NCHUNK * CHUNK, 8)
    pltpu.async_copy(x_hbm.at[idx_s.at[pl.ds(t0, TAIL)]],
                     rows.at[0, pl.ds(0, TAIL)], sem_g0)
    pltpu.make_async_copy(x_hbm.at[idx_s.at[pl.ds(t0, TAIL)]],
                          rows.at[0, pl.ds(0, TAIL)], sem_g0).wait()
    pltpu.async_copy(rows.at[0, pl.ds(0, TAIL)], acc.at[idx_dt.at[0]],
                     sem_s0, add=True)
    pltpu.make_async_copy(rows.at[0, pl.ds(0, TAIL)], acc.at[idx_dt.at[0]],
                          sem_s0).wait()
    scatter_wait(NCHUNK - 1, 1)
    plsc.subcore_barrier()

    pltpu.sync_copy(acc.at[pl.ds(row0, RPT)], out_hbm.at[c, pl.ds(row0, RPT)])

    @pl.when(s == NS - 1)
    def _():
        pltpu.sync_copy(acc.at[pl.ds(N - 16, 16)],
                        out_hbm.at[c, pl.ds(N - 16, 16)])


@functools.lru_cache(maxsize=None)
def _seg_sum_call():
    return pl.kernel(
        _seg_sum_body,
        out_type=jax.ShapeDtypeStruct((NC, N, D), jnp.float32),
        mesh=plsc.VectorSubcoreMesh(core_axis_name="c", subcore_axis_name="s"),
        scratch_types=[
            pltpu.VMEM((EPW,), jnp.int32),
            pltpu.VMEM((NCHUNK, CHUNK), jnp.int32),
            pltpu.VMEM((1, TAIL), jnp.int32),
            pltpu.VMEM((2, CHUNK, D), jnp.float32),
            pltpu.VMEM_SHARED((N, D), jnp.float32),
            pltpu.SemaphoreType.DMA,
            pltpu.SemaphoreType.DMA,
            pltpu.SemaphoreType.DMA,
            pltpu.SemaphoreType.DMA,
            pltpu.SemaphoreType.DMA,
        ],
    )


def _res_body(x_ref, wres_ref, res_ref):
    res_ref[...] = jnp.maximum(jnp.dot(x_ref[...], wres_ref[...],
                               preferred_element_type=jnp.float32), 0.0)


@functools.lru_cache(maxsize=None)
def _res_call():
    return pl.pallas_call(
        _res_body,
        grid=(NB,),
        in_specs=[
            pl.BlockSpec((BLK, D), lambda i: (i, 0)),
            pl.BlockSpec((D, D), lambda i: (0, 0)),
        ],
        out_specs=pl.BlockSpec((BLK, D), lambda i: (i, 0)),
        out_shape=jax.ShapeDtypeStruct((N, D), jnp.float32),
    )


def _fused_body(p_ref, x_ref, w_ref, wres_ref, g_ref, b_ref, out_ref,
                hbuf, stats):
    pp = pl.program_id(0)
    i = pl.program_id(1)

    @pl.when(pp == 0)
    def _():
        agg = p_ref[0] + p_ref[1]
        h = jnp.maximum(jnp.dot(agg, w_ref[...],
                                preferred_element_type=jnp.float32), 0.0)
        h = h + jnp.maximum(jnp.dot(x_ref[...], wres_ref[...],
                                    preferred_element_type=jnp.float32), 0.0)
        hbuf[pl.ds(i * BLK, BLK), :] = h

        @pl.when(i == 0)
        def _():
            stats[...] = jnp.zeros_like(stats)
        stats[0:1] += jnp.sum(h, axis=0, keepdims=True)
        stats[1:2] += jnp.sum(h * h, axis=0, keepdims=True)

    @pl.when(pp == 1)
    def _():
        h = hbuf[pl.ds(i * BLK, BLK), :]
        mean = stats[0:1] * (1.0 / N)
        var = stats[1:2] * (1.0 / N) - mean * mean
        inv = lax.rsqrt(var + 1e-5)
        out_ref[...] = (h - mean) * (inv * g_ref[...]) + b_ref[...]


@functools.lru_cache(maxsize=None)
def _fused_call():
    return pl.pallas_call(
        _fused_body,
        grid=(2, NB),
        in_specs=[
            pl.BlockSpec((NC, BLK, D),
                         lambda p, i: (0, jnp.where(p == 0, i, NB - 1), 0)),
            pl.BlockSpec((BLK, D),
                         lambda p, i: (jnp.where(p == 0, i, NB - 1), 0)),
            pl.BlockSpec((D, D), lambda p, i: (0, 0)),
            pl.BlockSpec((D, D), lambda p, i: (0, 0)),
            pl.BlockSpec((1, D), lambda p, i: (0, 0)),
            pl.BlockSpec((1, D), lambda p, i: (0, 0)),
        ],
        out_specs=pl.BlockSpec((BLK, D),
                               lambda p, i: (jnp.where(p == 0, 0, i), 0)),
        out_shape=jax.ShapeDtypeStruct((N, D), jnp.float32),
        scratch_shapes=[
            pltpu.VMEM((N, D), jnp.float32),
            pltpu.VMEM((2, D), jnp.float32),
        ],
    )


def _layer(x, src, dst_m, dst_t, w, wres, gamma, beta):
    p = _seg_sum_call()(x, src, dst_m, dst_t)
    return _fused_call()(p, x, w, wres,
                         gamma.reshape(1, D), beta.reshape(1, D))


def kernel(x, edge_index, W1, Wres1, gamma1, beta1, W2, Wres2, gamma2, beta2):
    src = edge_index[0]
    dst2 = edge_index[1].reshape(NW, EPW)
    dst_m = dst2[:, :NCHUNK * CHUNK].reshape(NW, NCHUNK, CHUNK)
    dst_t = dst2[:, NCHUNK * CHUNK:].reshape(NW, 1, TAIL)
    h = _layer(x, src, dst_m, dst_t, W1, Wres1, gamma1, beta1)
    h = _layer(h, src, dst_m, dst_t, W2, Wres2, gamma2, beta2)
    return h


# speedup vs baseline: 1.0670x; 1.0670x over previous
import functools

import jax
import jax.numpy as jnp
from jax import lax
from jax.experimental import pallas as pl
from jax.experimental.pallas import tpu as pltpu
from jax.experimental.pallas import tpu_sc as plsc

N = 10000
D = 128
E = 320000
NC = 2
NS = 16
NW = NC * NS
EPW = E // NW
CHUNK = 128
NCHUNK = EPW // CHUNK
TAIL = EPW - NCHUNK * CHUNK
RPT = 624
ZR = 48

BLK = 1000
NB = N // BLK


def _seg_sum_body(x_hbm, src_hbm, dstm_hbm, dstt_hbm, out_hbm,
                  idx_s, dstb0, dstb1, idx_dt, rows, acc,
                  sem_ip, sem_g0, sem_g1, sem_s0, sem_s1, sem_d0, sem_d1):
    c = lax.axis_index("c")
    s = lax.axis_index("s")
    wid = s * NC + c

    sem_g = (sem_g0, sem_g1)
    sem_s = (sem_s0, sem_s1)
    sem_d = (sem_d0, sem_d1)
    dstb = (dstb0, dstb1)

    def gather_start(j, b):
        pltpu.async_copy(x_hbm.at[idx_s.at[pl.ds(j * CHUNK, CHUNK)]],
                         rows.at[b], sem_g[b])

    def gather_wait(j, b):
        pltpu.make_async_copy(x_hbm.at[idx_s.at[pl.ds(j * CHUNK, CHUNK)]],
                              rows.at[b], sem_g[b]).wait()

    def scatter_start(j, b):
        pltpu.async_copy(rows.at[b], acc.at[dstb[b].at[0]], sem_s[b],
                         add=True)

    def scatter_wait(j, b):
        pltpu.make_async_copy(rows.at[b], acc.at[dstb[b].at[0]],
                              sem_s[b]).wait()

    def d_load(j, b):
        pltpu.async_copy(dstm_hbm.at[wid, j], dstb[b], sem_d[b])

    def d_wait(j, b):
        pltpu.make_async_copy(dstm_hbm.at[wid, j], dstb[b], sem_d[b]).wait()

    ip_s = pltpu.async_copy(
        src_hbm.at[pl.ds(pl.multiple_of(wid * EPW, 8), EPW)], idx_s, sem_ip)
    ip_t = pltpu.async_copy(dstt_hbm.at[wid], idx_dt, sem_ip)
    d_load(0, 0)

    def zb(i, _):
        r = i // 8
        col = (i % 8) * 16
        rows[0, r, pl.ds(col, 16)] = jnp.zeros((16,), jnp.float32)
        return 0
    lax.fori_loop(0, ZR * 8, zb, 0)

    row0 = s * RPT

    def zoff(i):
        return pl.ds(pl.multiple_of(row0 + i * ZR, 8), ZR)

    def zc(i, _):
        pltpu.async_copy(rows.at[0, pl.ds(0, ZR)], acc.at[zoff(i)], sem_ip)
        return 0
    lax.fori_loop(0, RPT // ZR, zc, 0)

    @pl.when(s == NS - 1)
    def _():
        pltpu.async_copy(rows.at[0, pl.ds(0, 16)],
                         acc.at[pl.ds(N - 16, 16)], sem_ip)

    def zd(i, _):
        pltpu.make_async_copy(rows.at[0, pl.ds(0, ZR)], acc.at[zoff(i)],
                              sem_ip).wait()
        return 0
    lax.fori_loop(0, RPT // ZR, zd, 0)

    @pl.when(s == NS - 1)
    def _():
        pltpu.make_async_copy(rows.at[0, pl.ds(0, 16)],
                              acc.at[pl.ds(N - 16, 16)], sem_ip).wait()
    ip_s.wait()
    ip_t.wait()
    plsc.subcore_barrier()
    gather_start(0, 0)

    def body(i, _):
        j0 = i * 2
        j1 = j0 + 1
        gather_wait(j0, 0)

        @pl.when(i > 0)
        def _():
            scatter_wait(j0 - 1, 1)
        gather_start(j1, 1)
        d_load(j1, 1)
        d_wait(j0, 0)
        scatter_start(j0, 0)
        gather_wait(j1, 1)
        scatter_wait(j0, 0)

        @pl.when(j0 + 2 < NCHUNK)
        def _():
            d_load(j0 + 2, 0)
            gather_start(j0 + 2, 0)
        d_wait(j1, 1)
        scatter_start(j1, 1)
        return 0
    lax.fori_loop(0, NCHUNK // 2, body, 0)
    t0 = pl.multiple_of(NCHUNK * CHUNK, 8)
    pltpu.async_copy(x_hbm.at[idx_s.at[pl.ds(t0, TAIL)]],
                     rows.at[0, pl.ds(0, TAIL)], sem_g0)
    pltpu.make_async_copy(x_hbm.at[idx_s.at[pl.ds(t0, TAIL)]],
                          rows.at[0, pl.ds(0, TAIL)], sem_g0).wait()
    pltpu.async_copy(rows.at[0, pl.ds(0, TAIL)], acc.at[idx_dt.at[0]],
                     sem_s0, add=True)
    pltpu.make_async_copy(rows.at[0, pl.ds(0, TAIL)], acc.at[idx_dt.at[0]],
                          sem_s0).wait()
    scatter_wait(NCHUNK - 1, 1)
    plsc.subcore_barrier()

    pltpu.sync_copy(acc.at[pl.ds(row0, RPT)], out_hbm.at[c, pl.ds(row0, RPT)])

    @pl.when(s == NS - 1)
    def _():
        pltpu.sync_copy(acc.at[pl.ds(N - 16, 16)],
                        out_hbm.at[c, pl.ds(N - 16, 16)])


@functools.lru_cache(maxsize=None)
def _seg_sum_call():
    return pl.kernel(
        _seg_sum_body,
        out_type=jax.ShapeDtypeStruct((NC, N, D), jnp.float32),
        mesh=plsc.VectorSubcoreMesh(core_axis_name="c", subcore_axis_name="s"),
        scratch_types=[
            pltpu.VMEM((EPW,), jnp.int32),
            pltpu.VMEM((1, CHUNK), jnp.int32),
            pltpu.VMEM((1, CHUNK), jnp.int32),
            pltpu.VMEM((1, TAIL), jnp.int32),
            pltpu.VMEM((2, CHUNK, D), jnp.float32),
            pltpu.VMEM_SHARED((N, D), jnp.float32),
            pltpu.SemaphoreType.DMA,
            pltpu.SemaphoreType.DMA,
            pltpu.SemaphoreType.DMA,
            pltpu.SemaphoreType.DMA,
            pltpu.SemaphoreType.DMA,
            pltpu.SemaphoreType.DMA,
            pltpu.SemaphoreType.DMA,
        ],
    )


def _res_body(x_ref, wres_ref, res_ref):
    res_ref[...] = jnp.maximum(jnp.dot(x_ref[...], wres_ref[...],
                               preferred_element_type=jnp.float32), 0.0)


@functools.lru_cache(maxsize=None)
def _res_call():
    return pl.pallas_call(
        _res_body,
        grid=(NB,),
        in_specs=[
            pl.BlockSpec((BLK, D), lambda i: (i, 0)),
            pl.BlockSpec((D, D), lambda i: (0, 0)),
        ],
        out_specs=pl.BlockSpec((BLK, D), lambda i: (i, 0)),
        out_shape=jax.ShapeDtypeStruct((N, D), jnp.float32),
    )


def _fused_body(p_ref, x_ref, w_ref, wres_ref, g_ref, b_ref, out_ref,
                hbuf, stats):
    pp = pl.program_id(0)
    i = pl.program_id(1)

    @pl.when(pp == 0)
    def _():
        agg = p_ref[0] + p_ref[1]
        h = jnp.maximum(jnp.dot(agg, w_ref[...],
                                preferred_element_type=jnp.float32), 0.0)
        h = h + jnp.maximum(jnp.dot(x_ref[...], wres_ref[...],
                                    preferred_element_type=jnp.float32), 0.0)
        hbuf[pl.ds(i * BLK, BLK), :] = h

        @pl.when(i == 0)
        def _():
            stats[...] = jnp.zeros_like(stats)
        stats[0:1] += jnp.sum(h, axis=0, keepdims=True)
        stats[1:2] += jnp.sum(h * h, axis=0, keepdims=True)

    @pl.when(pp == 1)
    def _():
        h = hbuf[pl.ds(i * BLK, BLK), :]
        mean = stats[0:1] * (1.0 / N)
        var = stats[1:2] * (1.0 / N) - mean * mean
        inv = lax.rsqrt(var + 1e-5)
        out_ref[...] = (h - mean) * (inv * g_ref[...]) + b_ref[...]


@functools.lru_cache(maxsize=None)
def _fused_call():
    return pl.pallas_call(
        _fused_body,
        grid=(2, NB),
        in_specs=[
            pl.BlockSpec((NC, BLK, D),
                         lambda p, i: (0, jnp.where(p == 0, i, NB - 1), 0)),
            pl.BlockSpec((BLK, D),
                         lambda p, i: (jnp.where(p == 0, i, NB - 1), 0)),
            pl.BlockSpec((D, D), lambda p, i: (0, 0)),
            pl.BlockSpec((D, D), lambda p, i: (0, 0)),
            pl.BlockSpec((1, D), lambda p, i: (0, 0)),
            pl.BlockSpec((1, D), lambda p, i: (0, 0)),
        ],
        out_specs=pl.BlockSpec((BLK, D),
                               lambda p, i: (jnp.where(p == 0, 0, i), 0)),
        out_shape=jax.ShapeDtypeStruct((N, D), jnp.float32),
        scratch_shapes=[
            pltpu.VMEM((N, D), jnp.float32),
            pltpu.VMEM((2, D), jnp.float32),
        ],
    )


def _layer(x, src, dst_m, dst_t, w, wres, gamma, beta):
    p = _seg_sum_call()(x, src, dst_m, dst_t)
    return _fused_call()(p, x, w, wres,
                         gamma.reshape(1, D), beta.reshape(1, D))


def kernel(x, edge_index, W1, Wres1, gamma1, beta1, W2, Wres2, gamma2, beta2):
    src = edge_index[0]
    dst2 = edge_index[1].reshape(NW, EPW)
    dst_m = dst2[:, :NCHUNK * CHUNK].reshape(NW, NCHUNK, 1, CHUNK)
    dst_t = dst2[:, NCHUNK * CHUNK:].reshape(NW, 1, TAIL)
    h = _layer(x, src, dst_m, dst_t, W1, Wres1, gamma1, beta1)
    h = _layer(h, src, dst_m, dst_t, W2, Wres2, gamma2, beta2)
    return h


# speedup vs baseline: 1.0701x; 1.0029x over previous
import functools

import jax
import jax.numpy as jnp
from jax import lax
from jax.experimental import pallas as pl
from jax.experimental.pallas import tpu as pltpu
from jax.experimental.pallas import tpu_sc as plsc

N = 10000
D = 128
E = 320000
NC = 2
NS = 16
NW = NC * NS
EPW = E // NW
CHUNK = 128
NCHUNK = EPW // CHUNK
TAIL = EPW - NCHUNK * CHUNK
RPT = 624
ZR = 48

BLK = 1000
NB = N // BLK


def _seg_sum_body(x_hbm, src_hbm, dstm_hbm, dstt_hbm, out_hbm,
                  idx_s, dstb0, dstb1, idx_dt, rows, acc,
                  sem_ip, sem_g0, sem_g1, sem_s0, sem_s1, sem_d0, sem_d1):
    c = lax.axis_index("c")
    s = lax.axis_index("s")
    wid = s * NC + c

    sem_g = (sem_g0, sem_g1)
    sem_s = (sem_s0, sem_s1)
    sem_d = (sem_d0, sem_d1)
    dstb = (dstb0, dstb1)

    def gather_start(j, b):
        pltpu.async_copy(x_hbm.at[idx_s.at[pl.ds(j * CHUNK, CHUNK)]],
                         rows.at[b], sem_g[b])

    def gather_wait(j, b):
        pltpu.make_async_copy(x_hbm.at[idx_s.at[pl.ds(j * CHUNK, CHUNK)]],
                              rows.at[b], sem_g[b]).wait()

    def scatter_start(j, b):
        pltpu.async_copy(rows.at[b], acc.at[dstb[b].at[0]], sem_s[b],
                         add=True)

    def scatter_wait(j, b):
        pltpu.make_async_copy(rows.at[b], acc.at[dstb[b].at[0]],
                              sem_s[b]).wait()

    def d_load(j, b):
        pltpu.async_copy(dstm_hbm.at[wid, j], dstb[b], sem_d[b])

    def d_wait(j, b):
        pltpu.make_async_copy(dstm_hbm.at[wid, j], dstb[b], sem_d[b]).wait()

    ip_s = pltpu.async_copy(
        src_hbm.at[pl.ds(pl.multiple_of(wid * EPW, 8), EPW)], idx_s, sem_ip)
    ip_t = pltpu.async_copy(dstt_hbm.at[wid], idx_dt, sem_ip)
    d_load(0, 0)

    def zb(i, _):
        r = i // 8
        col = (i % 8) * 16
        rows[0, r, pl.ds(col, 16)] = jnp.zeros((16,), jnp.float32)
        return 0
    lax.fori_loop(0, ZR * 8, zb, 0)

    row0 = s * RPT

    def zoff(i):
        return pl.ds(pl.multiple_of(row0 + i * ZR, 8), ZR)

    def zc(i, _):
        pltpu.async_copy(rows.at[0, pl.ds(0, ZR)], acc.at[zoff(i)], sem_ip)
        return 0
    lax.fori_loop(0, RPT // ZR, zc, 0)

    @pl.when(s == NS - 1)
    def _():
        pltpu.async_copy(rows.at[0, pl.ds(0, 16)],
                         acc.at[pl.ds(N - 16, 16)], sem_ip)

    def zd(i, _):
        pltpu.make_async_copy(rows.at[0, pl.ds(0, ZR)], acc.at[zoff(i)],
                              sem_ip).wait()
        return 0
    lax.fori_loop(0, RPT // ZR, zd, 0)

    @pl.when(s == NS - 1)
    def _():
        pltpu.make_async_copy(rows.at[0, pl.ds(0, 16)],
                              acc.at[pl.ds(N - 16, 16)], sem_ip).wait()
    ip_s.wait()
    ip_t.wait()
    plsc.subcore_barrier()
    gather_start(0, 0)

    def body(i, _):
        j0 = i * 2
        j1 = j0 + 1
        gather_wait(j0, 0)

        @pl.when(i > 0)
        def _():
            scatter_wait(j0 - 1, 1)
        gather_start(j1, 1)
        d_load(j1, 1)
        d_wait(j0, 0)
        scatter_start(j0, 0)
        gather_wait(j1, 1)
        scatter_wait(j0, 0)

        @pl.when(j0 + 2 < NCHUNK)
        def _():
            d_load(j0 + 2, 0)
            gather_start(j0 + 2, 0)
        d_wait(j1, 1)
        scatter_start(j1, 1)
        return 0
    lax.fori_loop(0, NCHUNK // 2, body, 0)
    t0 = pl.multiple_of(NCHUNK * CHUNK, 8)
    pltpu.async_copy(x_hbm.at[idx_s.at[pl.ds(t0, TAIL)]],
                     rows.at[0, pl.ds(0, TAIL)], sem_g0)
    pltpu.make_async_copy(x_hbm.at[idx_s.at[pl.ds(t0, TAIL)]],
                          rows.at[0, pl.ds(0, TAIL)], sem_g0).wait()
    pltpu.async_copy(rows.at[0, pl.ds(0, TAIL)], acc.at[idx_dt.at[0]],
                     sem_s0, add=True)
    pltpu.make_async_copy(rows.at[0, pl.ds(0, TAIL)], acc.at[idx_dt.at[0]],
                          sem_s0).wait()
    scatter_wait(NCHUNK - 1, 1)
    plsc.subcore_barrier()

    pltpu.sync_copy(acc.at[pl.ds(row0, RPT)], out_hbm.at[c, pl.ds(row0, RPT)])

    @pl.when(s == NS - 1)
    def _():
        pltpu.sync_copy(acc.at[pl.ds(N - 16, 16)],
                        out_hbm.at[c, pl.ds(N - 16, 16)])


@functools.lru_cache(maxsize=None)
def _seg_sum_call():
    return pl.kernel(
        _seg_sum_body,
        out_type=jax.ShapeDtypeStruct((NC, N, D), jnp.float32),
        mesh=plsc.VectorSubcoreMesh(core_axis_name="c", subcore_axis_name="s"),
        scratch_types=[
            pltpu.VMEM((EPW,), jnp.int32),
            pltpu.VMEM((1, CHUNK), jnp.int32),
            pltpu.VMEM((1, CHUNK), jnp.int32),
            pltpu.VMEM((1, TAIL), jnp.int32),
            pltpu.VMEM((2, CHUNK, D), jnp.float32),
            pltpu.VMEM_SHARED((N, D), jnp.float32),
            pltpu.SemaphoreType.DMA,
            pltpu.SemaphoreType.DMA,
            pltpu.SemaphoreType.DMA,
            pltpu.SemaphoreType.DMA,
            pltpu.SemaphoreType.DMA,
            pltpu.SemaphoreType.DMA,
            pltpu.SemaphoreType.DMA,
        ],
    )


def _fused_body(p_ref, x_ref, w_ref, wres_ref, g_ref, b_ref, out_ref,
                hbuf, stats):
    pp = pl.program_id(0)
    i = pl.program_id(1)

    @pl.when(pp == 0)
    def _():
        agg = p_ref[0] + p_ref[1]
        h = jnp.maximum(jnp.dot(agg, w_ref[...],
                                preferred_element_type=jnp.float32), 0.0)
        h = h + jnp.maximum(jnp.dot(x_ref[...], wres_ref[...],
                                    preferred_element_type=jnp.float32), 0.0)
        hbuf[pl.ds(i * BLK, BLK), :] = h

        @pl.when(i == 0)
        def _():
            stats[...] = jnp.zeros_like(stats)
        stats[0:1] += jnp.sum(h, axis=0, keepdims=True)
        stats[1:2] += jnp.sum(h * h, axis=0, keepdims=True)

    @pl.when(pp == 1)
    def _():
        h = hbuf[pl.ds(i * BLK, BLK), :]
        mean = stats[0:1] * (1.0 / N)
        var = stats[1:2] * (1.0 / N) - mean * mean
        inv = lax.rsqrt(var + 1e-5)
        out_ref[...] = (h - mean) * (inv * g_ref[...]) + b_ref[...]


@functools.lru_cache(maxsize=None)
def _fused_call():
    return pl.pallas_call(
        _fused_body,
        grid=(2, NB),
        in_specs=[
            pl.BlockSpec((NC, BLK, D),
                         lambda p, i: (0, jnp.where(p == 0, i, NB - 1), 0)),
            pl.BlockSpec((BLK, D),
                         lambda p, i: (jnp.where(p == 0, i, NB - 1), 0)),
            pl.BlockSpec((D, D), lambda p, i: (0, 0)),
            pl.BlockSpec((D, D), lambda p, i: (0, 0)),
            pl.BlockSpec((1, D), lambda p, i: (0, 0)),
            pl.BlockSpec((1, D), lambda p, i: (0, 0)),
        ],
        out_specs=pl.BlockSpec((BLK, D),
                               lambda p, i: (jnp.where(p == 0, 0, i), 0)),
        out_shape=jax.ShapeDtypeStruct((N, D), jnp.float32),
        scratch_shapes=[
            pltpu.VMEM((N, D), jnp.float32),
            pltpu.VMEM((2, D), jnp.float32),
        ],
    )


def _layer(x, src, dst_m, dst_t, w, wres, gamma, beta):
    p = _seg_sum_call()(x, src, dst_m, dst_t)
    return _fused_call()(p, x, w, wres,
                         gamma.reshape(1, D), beta.reshape(1, D))


def kernel(x, edge_index, W1, Wres1, gamma1, beta1, W2, Wres2, gamma2, beta2):
    src = edge_index[0]
    dst2 = edge_index[1].reshape(NW, EPW)
    dst_m = dst2[:, :NCHUNK * CHUNK].reshape(NW, NCHUNK, 1, CHUNK)
    dst_t = dst2[:, NCHUNK * CHUNK:].reshape(NW, 1, TAIL)
    h = _layer(x, src, dst_m, dst_t, W1, Wres1, gamma1, beta1)
    h = _layer(h, src, dst_m, dst_t, W2, Wres2, gamma2, beta2)
    return h
